# Initial kernel scaffold; baseline (speedup 1.0000x reference)
import jax
import jax.numpy as jnp
from jax.experimental import pallas as pl


def kernel(atomic, valence, formal_charge, aromatic, hybridization, radical_electrons, atomic_table, valence_table, hybridization_table):
    raise NotImplementedError("write your pallas kernel here")


# speedup vs baseline: 1.4384x; 1.4384x over previous
import functools

import jax
import jax.numpy as jnp
from jax import lax
from jax.experimental import pallas as pl
from jax.experimental.pallas import tpu as pltpu
from jax.experimental.pallas import tpu_sc as plsc

N = 100000
D = 128
OUT_D = 3 * D + 3
S = 80
NSTEPS = N // S
NW = 32
TAIL = OUT_D - 16

_mesh = plsc.VectorSubcoreMesh(core_axis_name="c", subcore_axis_name="s")


@functools.partial(
    pl.kernel,
    mesh=_mesh,
    out_type=jax.ShapeDtypeStruct((N, OUT_D), jnp.float32),
    scratch_types=[
        pltpu.VMEM((S,), jnp.int32),
        pltpu.VMEM((S,), jnp.int32),
        pltpu.VMEM((S,), jnp.int32),
        pltpu.VMEM((3 * S,), jnp.float32),
        pltpu.VMEM((S, OUT_D), jnp.float32),
        pltpu.SemaphoreType.DMA,
    ],
)
def _embed(atomic_h, valence_h, fc_h, ar_h, hyb_h, re_h, ta_h, tv_h, th_h,
           out_h, ia, iv, ih, pack, block, sem):
    wid = lax.axis_index("s") * 2 + lax.axis_index("c")
    nsteps_w = (NSTEPS - 1 - wid) // NW + 1
    ita = lax.iota(jnp.int32, 16)
    is14 = ita == 14
    is15 = ita == 15

    def body(i, carry):
        step = wid + NW * i
        base = step * S
        pltpu.sync_copy(atomic_h.at[pl.ds(base, S)], ia)
        pltpu.sync_copy(valence_h.at[pl.ds(base, S)], iv)
        pltpu.sync_copy(hyb_h.at[pl.ds(base, S)], ih)
        pltpu.sync_copy(fc_h.at[pl.ds(base, S)], pack.at[pl.ds(0, S)])
        pltpu.sync_copy(ar_h.at[pl.ds(base, S)], pack.at[pl.ds(S, S)])
        pltpu.sync_copy(re_h.at[pl.ds(base, S)], pack.at[pl.ds(2 * S, S)])
        ca = pltpu.async_copy(ta_h.at[ia], block.at[pl.ds(0, S), pl.ds(0, D)], sem)
        cv = pltpu.async_copy(tv_h.at[iv], block.at[pl.ds(0, S), pl.ds(D, D)], sem)

        def fgroup(g, c):
            fcv = pack[pl.ds(g * 16, 16)]
            arv = pack[pl.ds(S + g * 16, 16)]
            rev = pack[pl.ds(2 * S + g * 16, 16)]
            for j in range(16):
                fcb = jnp.full((16,), fcv[j], jnp.float32)
                arb = jnp.full((16,), arv[j], jnp.float32)
                reb = jnp.full((16,), rev[j], jnp.float32)
                tail = jnp.where(is14, arb, jnp.where(is15, reb, fcb))
                block[g * 16 + j, pl.ds(TAIL, 16)] = tail
            return c

        lax.fori_loop(0, S // 16, fgroup, 0)
        ch = pltpu.async_copy(th_h.at[ih], block.at[pl.ds(0, S), pl.ds(2 * D, D)], sem)
        ca.wait()
        cv.wait()
        ch.wait()
        pltpu.sync_copy(block, out_h.at[pl.ds(base, S)])
        return carry

    lax.fori_loop(0, nsteps_w, body, 0)


def kernel(atomic, valence, formal_charge, aromatic, hybridization,
           radical_electrons, atomic_table, valence_table, hybridization_table):
    return _embed(atomic, valence, formal_charge, aromatic, hybridization,
                  radical_electrons, atomic_table, valence_table,
                  hybridization_table)


# speedup vs baseline: 1.7981x; 1.2501x over previous
import functools

import jax
import jax.numpy as jnp
from jax import lax
from jax.experimental import pallas as pl
from jax.experimental.pallas import tpu as pltpu
from jax.experimental.pallas import tpu_sc as plsc

N = 100000
D = 128
OUT_D = 3 * D + 3
S = 80
NSTEPS = N // S
NW = 32
TAIL = OUT_D - 16

_mesh = plsc.VectorSubcoreMesh(core_axis_name="c", subcore_axis_name="s")


@functools.partial(
    pl.kernel,
    mesh=_mesh,
    out_type=jax.ShapeDtypeStruct((N, OUT_D), jnp.float32),
    scratch_types=[
        pltpu.VMEM((S,), jnp.int32),
        pltpu.VMEM((S,), jnp.int32),
        pltpu.VMEM((S,), jnp.int32),
        pltpu.VMEM((S,), jnp.int32),
        pltpu.VMEM((S,), jnp.int32),
        pltpu.VMEM((S,), jnp.int32),
        pltpu.VMEM((3 * S,), jnp.float32),
        pltpu.VMEM((3 * S,), jnp.float32),
        pltpu.VMEM((S, OUT_D), jnp.float32),
        pltpu.VMEM((S, OUT_D), jnp.float32),
        pltpu.SemaphoreType.DMA,
        pltpu.SemaphoreType.DMA,
        pltpu.SemaphoreType.DMA,
        pltpu.SemaphoreType.DMA,
        pltpu.SemaphoreType.DMA,
        pltpu.SemaphoreType.DMA,
    ],
)
def _embed(atomic_h, valence_h, fc_h, ar_h, hyb_h, re_h, ta_h, tv_h, th_h,
           out_h, ia0, ia1, iv0, iv1, ih0, ih1, pack0, pack1, block0, block1,
           sem_in0, sem_in1, sem_g0, sem_g1, sem_o0, sem_o1):
    wid = lax.axis_index("s") * 2 + lax.axis_index("c")
    nsteps_w = (NSTEPS - 1 - wid) // NW + 1
    ita = lax.iota(jnp.int32, 16)
    is14 = ita == 14
    is15 = ita == 15

    ias = (ia0, ia1)
    ivs = (iv0, iv1)
    ihs = (ih0, ih1)
    packs = (pack0, pack1)
    blocks = (block0, block1)
    sem_in = (sem_in0, sem_in1)
    sem_g = (sem_g0, sem_g1)
    sem_o = (sem_o0, sem_o1)

    def issue_in(step, b):
        base = step * S
        pltpu.async_copy(atomic_h.at[pl.ds(base, S)], ias[b], sem_in[b])
        pltpu.async_copy(valence_h.at[pl.ds(base, S)], ivs[b], sem_in[b])
        pltpu.async_copy(hyb_h.at[pl.ds(base, S)], ihs[b], sem_in[b])
        pltpu.async_copy(fc_h.at[pl.ds(base, S)], packs[b].at[pl.ds(0, S)], sem_in[b])
        pltpu.async_copy(ar_h.at[pl.ds(base, S)], packs[b].at[pl.ds(S, S)], sem_in[b])
        pltpu.async_copy(re_h.at[pl.ds(base, S)], packs[b].at[pl.ds(2 * S, S)], sem_in[b])

    def wait_in(b):
        pltpu.make_async_copy(atomic_h.at[pl.ds(0, S)], ias[b], sem_in[b]).wait()
        pltpu.make_async_copy(valence_h.at[pl.ds(0, S)], ivs[b], sem_in[b]).wait()
        pltpu.make_async_copy(hyb_h.at[pl.ds(0, S)], ihs[b], sem_in[b]).wait()
        pltpu.make_async_copy(fc_h.at[pl.ds(0, S)], packs[b].at[pl.ds(0, S)], sem_in[b]).wait()
        pltpu.make_async_copy(ar_h.at[pl.ds(0, S)], packs[b].at[pl.ds(S, S)], sem_in[b]).wait()
        pltpu.make_async_copy(re_h.at[pl.ds(0, S)], packs[b].at[pl.ds(2 * S, S)], sem_in[b]).wait()

    def wait_out(b):
        pltpu.make_async_copy(blocks[b], out_h.at[pl.ds(0, S)], sem_o[b]).wait()

    def do_step(j, b):
        step = wid + NW * j
        base = step * S
        blk = blocks[b]
        pack = packs[b]
        wait_in(b)

        @pl.when(j + 1 < nsteps_w)
        def _():
            issue_in(wid + NW * (j + 1), 1 - b)

        @pl.when(j >= 2)
        def _():
            wait_out(b)

        ca = pltpu.async_copy(ta_h.at[ias[b]], blk.at[pl.ds(0, S), pl.ds(0, D)], sem_g[b])
        cv = pltpu.async_copy(tv_h.at[ivs[b]], blk.at[pl.ds(0, S), pl.ds(D, D)], sem_g[b])

        def fgroup(g, c):
            fcv = pack[pl.ds(g * 16, 16)]
            arv = pack[pl.ds(S + g * 16, 16)]
            rev = pack[pl.ds(2 * S + g * 16, 16)]
            for j16 in range(16):
                fcb = jnp.full((16,), fcv[j16], jnp.float32)
                arb = jnp.full((16,), arv[j16], jnp.float32)
                reb = jnp.full((16,), rev[j16], jnp.float32)
                tail = jnp.where(is14, arb, jnp.where(is15, reb, fcb))
                blk[g * 16 + j16, pl.ds(TAIL, 16)] = tail
            return c

        lax.fori_loop(0, S // 16, fgroup, 0)
        ch = pltpu.async_copy(th_h.at[ihs[b]], blk.at[pl.ds(0, S), pl.ds(2 * D, D)], sem_g[b])
        ca.wait()
        cv.wait()
        ch.wait()
        pltpu.async_copy(blk, out_h.at[pl.ds(base, S)], sem_o[b])

    @pl.when(nsteps_w > 0)
    def _():
        issue_in(wid, 0)

    def pair_body(k, c):
        for b in (0, 1):
            j = 2 * k + b

            @pl.when(j < nsteps_w)
            def _():
                do_step(j, b)

        return c

    lax.fori_loop(0, (nsteps_w + 1) // 2, pair_body, 0)
    for b in (0, 1):
        @pl.when(nsteps_w > b)
        def _():
            wait_out(b)


def kernel(atomic, valence, formal_charge, aromatic, hybridization,
           radical_electrons, atomic_table, valence_table, hybridization_table):
    return _embed(atomic, valence, formal_charge, aromatic, hybridization,
                  radical_electrons, atomic_table, valence_table,
                  hybridization_table)


# speedup vs baseline: 3.8710x; 2.1528x over previous
import functools

import jax
import jax.numpy as jnp
from jax import lax
from jax.experimental import pallas as pl
from jax.experimental.pallas import tpu as pltpu
from jax.experimental.pallas import tpu_sc as plsc

N = 100000
D = 128
OUT_D = 3 * D + 3
S = 80
NSTEPS = N // S
NW = 32
TAIL = OUT_D - 16
NUM_ATOMIC = 119
NUM_VALENCE = 8
NUM_HYBRID = 8

_mesh = plsc.VectorSubcoreMesh(core_axis_name="c", subcore_axis_name="s")


@functools.partial(
    pl.kernel,
    mesh=_mesh,
    compiler_params=pltpu.CompilerParams(needs_layout_passes=False),
    out_type=jax.ShapeDtypeStruct((N, OUT_D), jnp.float32),
    scratch_types=[
        pltpu.VMEM((NUM_ATOMIC, D), jnp.float32),
        pltpu.VMEM((NUM_VALENCE, D), jnp.float32),
        pltpu.VMEM((NUM_HYBRID, D), jnp.float32),
        pltpu.VMEM((S,), jnp.int32),
        pltpu.VMEM((S,), jnp.int32),
        pltpu.VMEM((S,), jnp.int32),
        pltpu.VMEM((S,), jnp.int32),
        pltpu.VMEM((S,), jnp.int32),
        pltpu.VMEM((S,), jnp.int32),
        pltpu.VMEM((3 * S,), jnp.float32),
        pltpu.VMEM((3 * S,), jnp.float32),
        pltpu.VMEM((S, OUT_D), jnp.float32),
        pltpu.VMEM((S, OUT_D), jnp.float32),
        pltpu.SemaphoreType.DMA,
        pltpu.SemaphoreType.DMA,
        pltpu.SemaphoreType.DMA,
        pltpu.SemaphoreType.DMA,
    ],
)
def _embed(atomic_h, valence_h, fc_h, ar_h, hyb_h, re_h, ta_h, tv_h, th_h,
           out_h, tab_a, tab_v, tab_h, ia0, ia1, iv0, iv1, ih0, ih1,
           pack0, pack1, block0, block1, sem_in0, sem_in1, sem_o0, sem_o1):
    wid = lax.axis_index("s") * 2 + lax.axis_index("c")
    nsteps_w = (NSTEPS - 1 - wid) // NW + 1
    ita = lax.iota(jnp.int32, 16)
    is14 = ita == 14
    is15 = ita == 15

    ias = (ia0, ia1)
    ivs = (iv0, iv1)
    ihs = (ih0, ih1)
    packs = (pack0, pack1)
    blocks = (block0, block1)
    sem_in = (sem_in0, sem_in1)
    sem_o = (sem_o0, sem_o1)

    pltpu.sync_copy(ta_h, tab_a)
    pltpu.sync_copy(tv_h, tab_v)
    pltpu.sync_copy(th_h, tab_h)

    def issue_in(step, b):
        base = step * S
        pltpu.async_copy(atomic_h.at[pl.ds(base, S)], ias[b], sem_in[b])
        pltpu.async_copy(valence_h.at[pl.ds(base, S)], ivs[b], sem_in[b])
        pltpu.async_copy(hyb_h.at[pl.ds(base, S)], ihs[b], sem_in[b])
        pltpu.async_copy(fc_h.at[pl.ds(base, S)], packs[b].at[pl.ds(0, S)], sem_in[b])
        pltpu.async_copy(ar_h.at[pl.ds(base, S)], packs[b].at[pl.ds(S, S)], sem_in[b])
        pltpu.async_copy(re_h.at[pl.ds(base, S)], packs[b].at[pl.ds(2 * S, S)], sem_in[b])

    def wait_in(b):
        pltpu.make_async_copy(atomic_h.at[pl.ds(0, S)], ias[b], sem_in[b]).wait()
        pltpu.make_async_copy(valence_h.at[pl.ds(0, S)], ivs[b], sem_in[b]).wait()
        pltpu.make_async_copy(hyb_h.at[pl.ds(0, S)], ihs[b], sem_in[b]).wait()
        pltpu.make_async_copy(fc_h.at[pl.ds(0, S)], packs[b].at[pl.ds(0, S)], sem_in[b]).wait()
        pltpu.make_async_copy(ar_h.at[pl.ds(0, S)], packs[b].at[pl.ds(S, S)], sem_in[b]).wait()
        pltpu.make_async_copy(re_h.at[pl.ds(0, S)], packs[b].at[pl.ds(2 * S, S)], sem_in[b]).wait()

    def wait_out(b):
        pltpu.make_async_copy(blocks[b], out_h.at[pl.ds(0, S)], sem_o[b]).wait()

    def do_step(j, b):
        step = wid + NW * j
        base = step * S
        blk = blocks[b]
        pack = packs[b]
        wait_in(b)

        @pl.when(j + 1 < nsteps_w)
        def _():
            issue_in(wid + NW * (j + 1), 1 - b)

        @pl.when(j >= 2)
        def _():
            wait_out(b)

        def fgroup(g, c):
            iav = ias[b][pl.ds(g * 16, 16)]
            ivv = ivs[b][pl.ds(g * 16, 16)]
            ihv = ihs[b][pl.ds(g * 16, 16)]
            fcv = pack[pl.ds(g * 16, 16)]
            arv = pack[pl.ds(S + g * 16, 16)]
            rev = pack[pl.ds(2 * S + g * 16, 16)]
            for j16 in range(16):
                r = g * 16 + j16
                for tab, idxv, band in ((tab_a, iav, 0), (tab_v, ivv, D),
                                        (tab_h, ihv, 2 * D)):
                    rowv = jnp.full((16,), idxv[j16], jnp.int32)
                    for c16 in range(8):
                        vals = plsc.load_gather(tab, [rowv, c16 * 16 + ita])
                        blk[r, pl.ds(band + c16 * 16, 16)] = vals
                fcb = jnp.full((16,), fcv[j16], jnp.float32)
                arb = jnp.full((16,), arv[j16], jnp.float32)
                reb = jnp.full((16,), rev[j16], jnp.float32)
                tail = jnp.where(is14, arb, jnp.where(is15, reb, fcb))
                blk[r, pl.ds(TAIL, 16)] = tail
            return c

        lax.fori_loop(0, S // 16, fgroup, 0)
        pltpu.async_copy(blk, out_h.at[pl.ds(base, S)], sem_o[b])

    @pl.when(nsteps_w > 0)
    def _():
        issue_in(wid, 0)

    def pair_body(k, c):
        for b in (0, 1):
            j = 2 * k + b

            @pl.when(j < nsteps_w)
            def _():
                do_step(j, b)

        return c

    lax.fori_loop(0, (nsteps_w + 1) // 2, pair_body, 0)
    for b in (0, 1):
        @pl.when(nsteps_w > b)
        def _():
            wait_out(b)


def kernel(atomic, valence, formal_charge, aromatic, hybridization,
           radical_electrons, atomic_table, valence_table, hybridization_table):
    return _embed(atomic, valence, formal_charge, aromatic, hybridization,
                  radical_electrons, atomic_table, valence_table,
                  hybridization_table)
